# baseline (speedup 1.0000x reference)
import functools

import jax
import jax.numpy as jnp
from jax import lax
from jax.experimental import pallas as pl
from jax.experimental.pallas import tpu as pltpu
from jax.experimental.pallas import tpu_sc as plsc


def _make_sc_pool(B, N_IN, C, NNZP):
    HALF = 10000
    SLAB = 10240
    STRIPE = SLAB // 16
    CH = 64
    PT = NNZP // 16
    NCHUNK = PT // CH
    NSLAB = B
    NV16 = PT // 16
    mesh = plsc.VectorSubcoreMesh(core_axis_name="c", subcore_axis_name="s")

    @functools.partial(
        pl.kernel, mesh=mesh,
        out_type=jax.ShapeDtypeStruct((B, 2, HALF, C), jnp.float32),
        scratch_types=[
            pltpu.VMEM((PT,), jnp.int32),
            pltpu.VMEM((PT,), jnp.float32),
            pltpu.VMEM((PT,), jnp.int32),
            pltpu.VMEM((CH,), jnp.int32),
            pltpu.VMEM((CH,), jnp.int32),
            pltpu.VMEM((CH,), jnp.int32),
            pltpu.VMEM((2, CH, C), jnp.float32),
            pltpu.VMEM((2, CH, 16), jnp.float32),
            pltpu.VMEM((32, C), jnp.float32),
            pltpu.VMEM_SHARED((SLAB, C), jnp.float32),
            pltpu.SemaphoreType.DMA((2,)),
            pltpu.SemaphoreType.DMA((2,)),
        ],
    )
    def sc_pool(x_hbm, col_hbm, row_hbm, val_hbm, vx_hbm, pooled_hbm,
                col_v, val_v, row_v, gidxa_v, gidxb_v, rsm_v,
                xbuf, vxbuf, zbuf, slab, sem, sem2):
        core = lax.axis_index("c")
        tid = lax.axis_index("s")
        nbase = tid * PT
        pltpu.sync_copy(col_hbm.at[pl.ds(nbase, PT)], col_v)
        pltpu.sync_copy(val_hbm.at[pl.ds(nbase, PT)], val_v)
        pltpu.sync_copy(row_hbm.at[pl.ds(nbase, PT)], row_v)

        def z16(j, _):
            zbuf[j // (C // 16), pl.ds((j % (C // 16)) * 16, 16)] = (
                jnp.zeros((16,), jnp.float32))
            return 0
        lax.fori_loop(0, (32 * C) // 16, z16, 0)

        def slab_body(sl, _):
            b = core * (B // 2) + sl // 2
            h = sl % 2

            xoff = b * N_IN

            def zc(i, _):
                pltpu.sync_copy(zbuf, slab.at[pl.ds(tid * STRIPE + i * 32, 32)])
                return 0
            lax.fori_loop(0, STRIPE // 32, zc, 0)
            plsc.subcore_barrier()

            def fire(c):
                @pl.when(c % 2 == 0)
                def _():
                    for k in range(CH // 16):
                        gidxa_v[pl.ds(k * 16, 16)] = (
                            col_v[pl.ds(c * CH + k * 16, 16)] + xoff)
                    pltpu.async_copy(x_hbm.at[gidxa_v],
                                     xbuf.at[c % 2], sem.at[c % 2])
                    pltpu.async_copy(
                        vx_hbm.at[pl.ds((tid * NCHUNK + c) * CH, CH)],
                        vxbuf.at[c % 2], sem2.at[c % 2])

                @pl.when(c % 2 == 1)
                def _():
                    for k in range(CH // 16):
                        gidxb_v[pl.ds(k * 16, 16)] = (
                            col_v[pl.ds(c * CH + k * 16, 16)] + xoff)
                    pltpu.async_copy(x_hbm.at[gidxb_v],
                                     xbuf.at[c % 2], sem.at[c % 2])
                    pltpu.async_copy(
                        vx_hbm.at[pl.ds((tid * NCHUNK + c) * CH, CH)],
                        vxbuf.at[c % 2], sem2.at[c % 2])

            fire(0)

            def chunk_body(c, _):
                pb = c % 2

                @pl.when(c + 1 < NCHUNK)
                def _():
                    fire(c + 1)

                pltpu.make_async_copy(x_hbm.at[pl.ds(0, CH)],
                                      xbuf.at[pb], sem.at[pb]).wait()
                pltpu.make_async_copy(vx_hbm.at[pl.ds(0, CH)],
                                      vxbuf.at[pb], sem2.at[pb]).wait()

                roff = h * HALF

                def scale_grp(grp, _):
                    off = c * CH + grp * 16
                    rr = row_v[pl.ds(off, 16)] - roff
                    ok = (rr >= 0) & (rr < HALF)
                    rsm_v[pl.ds(grp * 16, 16)] = jnp.where(ok, rr, HALF)
                    for l in range(16):
                        i = grp * 16 + l
                        v16 = vxbuf[pb, i, pl.ds(0, 16)]
                        for j in range(C // 16):
                            xbuf[pb, i, pl.ds(j * 16, 16)] = (
                                xbuf[pb, i, pl.ds(j * 16, 16)] * v16)
                    return 0
                lax.fori_loop(0, CH // 16, scale_grp, 0)

                pltpu.sync_copy(xbuf.at[pb], slab.at[rsm_v], add=True)
                return 0

            lax.fori_loop(0, NCHUNK, chunk_body, 0)
            plsc.subcore_barrier()

            @pl.when(tid < 15)
            def _():
                pltpu.sync_copy(
                    slab.at[pl.ds(tid * STRIPE, STRIPE)],
                    pooled_hbm.at[b, h, pl.ds(tid * STRIPE, STRIPE)])

            @pl.when(tid == 15)
            def _():
                pltpu.sync_copy(
                    slab.at[pl.ds(15 * STRIPE, HALF - 15 * STRIPE)],
                    pooled_hbm.at[b, h, pl.ds(15 * STRIPE,
                                              HALF - 15 * STRIPE)])
            return 0

        lax.fori_loop(0, NSLAB, slab_body, 0)

    return sc_pool


def _mm_body(p_ref, w_ref, z_ref):
    for s in range(w_ref.shape[0]):
        z_ref[s] = jnp.dot(p_ref[0], w_ref[s],
                           preferred_element_type=jnp.float32)


def _matmul_z(pooled, Ws):
    B, N_OUT, C = pooled.shape
    S, _, C_OUT = Ws.shape
    TILE = 2000
    return pl.pallas_call(
        _mm_body,
        grid=(B, N_OUT // TILE),
        in_specs=[
            pl.BlockSpec((1, TILE, C), lambda b, u: (b, u, 0)),
            pl.BlockSpec((S, C, C_OUT), lambda b, u: (0, 0, 0)),
        ],
        out_specs=pl.BlockSpec((S, TILE, C_OUT),
                               lambda b, u: (0, u, b)),
        out_shape=jax.ShapeDtypeStruct((S, N_OUT, B * C_OUT), jnp.float32),
    )(pooled, Ws)


def _make_sc_gather(S, N_OUT, B, C, NW, VPW, NCH, NTI):
    ROW = B * C
    VC = 16
    T = NCH * S
    mesh = plsc.VectorSubcoreMesh(core_axis_name="c", subcore_axis_name="s")

    @functools.partial(
        pl.kernel, mesh=mesh,
        out_type=jax.ShapeDtypeStruct((N_OUT * B, C), jnp.float32),
        scratch_types=[
            pltpu.VMEM((NTI, 8, 128), jnp.int32),
            pltpu.VMEM((2, VC, ROW), jnp.float32),
            pltpu.VMEM((VC * B, C), jnp.float32),
            pltpu.VMEM((1, ROW), jnp.float32),
            pltpu.SemaphoreType.DMA((2,)),
        ],
    )
    def sc_gather(z_hbm, idx_hbm, bias_hbm, out_hbm,
                  idx_v, zbuf, acc, bias_v, sem):
        wid = lax.axis_index("s") * 2 + lax.axis_index("c")
        pltpu.sync_copy(idx_hbm.at[wid], idx_v)
        pltpu.sync_copy(bias_hbm, bias_v)

        def fire(t):
            ireg = idx_v[t // 64, (t % 64) // 8, pl.ds((t % 8) * 16, 16)]
            pltpu.async_copy(z_hbm.at[ireg],
                             zbuf.at[t % 2], sem.at[t % 2])

        fire(0)

        def step(t, _):
            c, s = t // S, t % S
            pb = t % 2

            pltpu.make_async_copy(z_hbm.at[pl.ds(0, VC)],
                                  zbuf.at[pb], sem.at[pb]).wait()

            @pl.when(t + 1 < T)
            def _():
                fire(t + 1)

            NJ = ROW // 16

            @pl.when(s == 0)
            def _():
                def cp(j, _):
                    o = j * 16
                    for i in range(VC):
                        acc[i * 16 + j // 8, pl.ds((j % 8) * 16, 16)] = (
                            zbuf[pb, i, pl.ds(o, 16)])
                    return 0
                lax.fori_loop(0, NJ, cp, 0)

            @pl.when((s > 0) & (s < S - 1))
            def _():
                def ad(j, _):
                    for u in range(2):
                        jj = j * 2 + u
                        o = jj * 16
                        for i in range(VC):
                            plsc.addupdate(
                                acc.at[i * 16 + jj // 8,
                                       pl.ds((jj % 8) * 16, 16)],
                                zbuf[pb, i, pl.ds(o, 16)])
                    return 0
                lax.fori_loop(0, NJ // 2, ad, 0)

            @pl.when(s == S - 1)
            def _():
                def fin(j, _):
                    o = j * 16
                    bv = bias_v[0, pl.ds(o, 16)]
                    for i in range(VC):
                        r = i * 16 + j // 8
                        ds = pl.ds((j % 8) * 16, 16)
                        acc[r, ds] = jnp.maximum(
                            acc[r, ds] + zbuf[pb, i, pl.ds(o, 16)] + bv, 0.0)
                    return 0
                lax.fori_loop(0, NJ, fin, 0)
                v0 = wid * VPW + jnp.minimum(c * VC, VPW - VC)
                pltpu.sync_copy(acc, out_hbm.at[pl.ds(v0 * B, VC * B)])
            return 0

        lax.fori_loop(0, T, step, 0)

    return sc_gather


def kernel(x, trans_row, trans_col, trans_val, spiral_idx, W, b):
    B, N_IN, C = x.shape
    N_OUT, S = spiral_idx.shape
    C_OUT = W.shape[1]
    ROW = B * C_OUT
    NW = 32
    VC = 16
    VPW = N_OUT // NW
    NCH = -(-VPW // VC)
    T = NCH * S
    T_PAD = -(-T // 64) * 64
    NTI = T_PAD // 64

    NNZ = trans_row.shape[0]
    NNZP = -(-NNZ // 2048) * 2048
    pad = NNZP - NNZ
    colp = jnp.pad(trans_col.astype(jnp.int32), (0, pad))
    rowp = jnp.pad(trans_row.astype(jnp.int32), (0, pad))
    valp = jnp.pad(trans_val.astype(jnp.float32), (0, pad))
    x2d = x.reshape(B * N_IN, C)
    vxp = jnp.broadcast_to(valp[:, None], (NNZP, 16))
    pool = _make_sc_pool(B, N_IN, C, NNZP)
    pooled = pool(x2d, colp, rowp, valp, vxp).reshape(B, N_OUT, C)

    Ws = W.reshape(S, C, C_OUT).astype(jnp.bfloat16)
    Z = _matmul_z(pooled.astype(jnp.bfloat16), Ws)
    z2d = Z.reshape(S * N_OUT, ROW)

    starts = jnp.minimum(jnp.arange(NCH, dtype=jnp.int32) * VC, VPW - VC)
    vglob = (jnp.arange(NW, dtype=jnp.int32)[:, None, None] * VPW
             + starts[None, :, None]
             + jnp.arange(VC, dtype=jnp.int32)[None, None, :])
    idx_g = spiral_idx.astype(jnp.int32)[vglob.reshape(-1)]
    idx_g = (idx_g + jnp.arange(S, dtype=jnp.int32)[None, :] * N_OUT)
    idx_wp = (idx_g.reshape(NW, NCH, VC, S)
              .transpose(0, 1, 3, 2)
              .reshape(NW, T, VC))
    idx_wp = jnp.pad(idx_wp, ((0, 0), (0, T_PAD - T), (0, 0)))
    idx_wp = idx_wp.reshape(NW, NTI, 8, 128)
    bias_row = jnp.tile(b.astype(jnp.float32), B)[None, :]

    sc = _make_sc_gather(S, N_OUT, B, C_OUT, NW, VPW, NCH, NTI)
    out2 = sc(z2d, idx_wp, bias_row)
    return out2.reshape(N_OUT, B, C_OUT).transpose(1, 0, 2)


# speedup vs baseline: 1.0239x; 1.0239x over previous
import functools

import jax
import jax.numpy as jnp
from jax import lax
from jax.experimental import pallas as pl
from jax.experimental.pallas import tpu as pltpu
from jax.experimental.pallas import tpu_sc as plsc


def _make_sc_pool(B, N_IN, C, NNZP):
    HALF = 10000
    SLAB = 10240
    STRIPE = SLAB // 16
    CH = 64
    PT = NNZP // 16
    NCHUNK = PT // CH
    NSLAB = B
    NV16 = PT // 16
    mesh = plsc.VectorSubcoreMesh(core_axis_name="c", subcore_axis_name="s")

    @functools.partial(
        pl.kernel, mesh=mesh,
        out_type=jax.ShapeDtypeStruct((B, 2, HALF, C), jnp.float32),
        scratch_types=[
            pltpu.VMEM((PT,), jnp.int32),
            pltpu.VMEM((PT,), jnp.float32),
            pltpu.VMEM((PT,), jnp.int32),
            pltpu.VMEM((CH,), jnp.int32),
            pltpu.VMEM((CH,), jnp.int32),
            pltpu.VMEM((CH,), jnp.int32),
            pltpu.VMEM((2, CH, C), jnp.float32),
            pltpu.VMEM((2, CH, 16), jnp.float32),
            pltpu.VMEM((32, C), jnp.float32),
            pltpu.VMEM_SHARED((SLAB, C), jnp.float32),
            pltpu.SemaphoreType.DMA((2,)),
            pltpu.SemaphoreType.DMA((2,)),
        ],
    )
    def sc_pool(x_hbm, col_hbm, row_hbm, val_hbm, vx_hbm, pooled_hbm,
                col_v, val_v, row_v, gidxa_v, gidxb_v, rsm_v,
                xbuf, vxbuf, zbuf, slab, sem, sem2):
        core = lax.axis_index("c")
        tid = lax.axis_index("s")
        nbase = tid * PT
        pltpu.sync_copy(col_hbm.at[pl.ds(nbase, PT)], col_v)
        pltpu.sync_copy(val_hbm.at[pl.ds(nbase, PT)], val_v)
        pltpu.sync_copy(row_hbm.at[pl.ds(nbase, PT)], row_v)

        def z16(j, _):
            zbuf[j // (C // 16), pl.ds((j % (C // 16)) * 16, 16)] = (
                jnp.zeros((16,), jnp.float32))
            return 0
        lax.fori_loop(0, (32 * C) // 16, z16, 0)

        def slab_body(sl, _):
            b = core * (B // 2) + sl // 2
            h = sl % 2

            xoff = b * N_IN

            def zc(i, _):
                pltpu.sync_copy(zbuf, slab.at[pl.ds(tid * STRIPE + i * 32, 32)])
                return 0
            lax.fori_loop(0, STRIPE // 32, zc, 0)
            plsc.subcore_barrier()

            def fire(c):
                @pl.when(c % 2 == 0)
                def _():
                    for k in range(CH // 16):
                        gidxa_v[pl.ds(k * 16, 16)] = (
                            col_v[pl.ds(c * CH + k * 16, 16)] + xoff)
                    pltpu.async_copy(x_hbm.at[gidxa_v],
                                     xbuf.at[c % 2], sem.at[c % 2])
                    pltpu.async_copy(
                        vx_hbm.at[pl.ds((tid * NCHUNK + c) * CH, CH)],
                        vxbuf.at[c % 2], sem2.at[c % 2])

                @pl.when(c % 2 == 1)
                def _():
                    for k in range(CH // 16):
                        gidxb_v[pl.ds(k * 16, 16)] = (
                            col_v[pl.ds(c * CH + k * 16, 16)] + xoff)
                    pltpu.async_copy(x_hbm.at[gidxb_v],
                                     xbuf.at[c % 2], sem.at[c % 2])
                    pltpu.async_copy(
                        vx_hbm.at[pl.ds((tid * NCHUNK + c) * CH, CH)],
                        vxbuf.at[c % 2], sem2.at[c % 2])

            fire(0)

            def chunk_body(c, _):
                pb = c % 2

                @pl.when(c + 1 < NCHUNK)
                def _():
                    fire(c + 1)

                pltpu.make_async_copy(x_hbm.at[pl.ds(0, CH)],
                                      xbuf.at[pb], sem.at[pb]).wait()
                pltpu.make_async_copy(vx_hbm.at[pl.ds(0, CH)],
                                      vxbuf.at[pb], sem2.at[pb]).wait()

                roff = h * HALF

                def scale_grp(grp, _):
                    off = c * CH + grp * 16
                    rr = row_v[pl.ds(off, 16)] - roff
                    ok = (rr >= 0) & (rr < HALF)
                    rsm_v[pl.ds(grp * 16, 16)] = jnp.where(ok, rr, HALF)
                    for l in range(16):
                        i = grp * 16 + l
                        v16 = vxbuf[pb, i, pl.ds(0, 16)]
                        for j in range(C // 16):
                            xbuf[pb, i, pl.ds(j * 16, 16)] = (
                                xbuf[pb, i, pl.ds(j * 16, 16)] * v16)
                    return 0
                lax.fori_loop(0, CH // 16, scale_grp, 0)

                pltpu.sync_copy(xbuf.at[pb], slab.at[rsm_v], add=True)
                return 0

            lax.fori_loop(0, NCHUNK, chunk_body, 0)
            plsc.subcore_barrier()

            @pl.when(tid < 15)
            def _():
                pltpu.sync_copy(
                    slab.at[pl.ds(tid * STRIPE, STRIPE)],
                    pooled_hbm.at[b, h, pl.ds(tid * STRIPE, STRIPE)])

            @pl.when(tid == 15)
            def _():
                pltpu.sync_copy(
                    slab.at[pl.ds(15 * STRIPE, HALF - 15 * STRIPE)],
                    pooled_hbm.at[b, h, pl.ds(15 * STRIPE,
                                              HALF - 15 * STRIPE)])
            return 0

        lax.fori_loop(0, NSLAB, slab_body, 0)

    return sc_pool


def _mm_body(p_ref, w_ref, z_ref):
    for s in range(w_ref.shape[0]):
        z_ref[s] = jnp.dot(p_ref[0], w_ref[s],
                           preferred_element_type=jnp.float32)


def _matmul_z(pooled, Ws):
    B, N_OUT, C = pooled.shape
    S, _, C_OUT = Ws.shape
    TILE = 2000
    return pl.pallas_call(
        _mm_body,
        grid=(B, N_OUT // TILE),
        in_specs=[
            pl.BlockSpec((1, TILE, C), lambda b, u: (b, u, 0)),
            pl.BlockSpec((S, C, C_OUT), lambda b, u: (0, 0, 0)),
        ],
        out_specs=pl.BlockSpec((S, TILE, C_OUT),
                               lambda b, u: (0, u, b)),
        out_shape=jax.ShapeDtypeStruct((S, N_OUT, B * C_OUT), jnp.float32),
    )(pooled, Ws)


def _make_sc_gather(S, N_OUT, B, C, NW, VPW, NCH, NTI):
    ROW = B * C
    VC = 16
    T = NCH * S
    mesh = plsc.VectorSubcoreMesh(core_axis_name="c", subcore_axis_name="s")

    @functools.partial(
        pl.kernel, mesh=mesh,
        out_type=jax.ShapeDtypeStruct((N_OUT * B, C), jnp.float32),
        scratch_types=[
            pltpu.VMEM((NTI, 8, 128), jnp.int32),
            pltpu.VMEM((2, VC, ROW), jnp.float32),
            pltpu.VMEM((VC * B, C), jnp.float32),
            pltpu.VMEM((1, ROW), jnp.float32),
            pltpu.SemaphoreType.DMA((2,)),
        ],
    )
    def sc_gather(z_hbm, idx_hbm, bias_hbm, out_hbm,
                  idx_v, zbuf, acc, bias_v, sem):
        wid = lax.axis_index("s") * 2 + lax.axis_index("c")
        pltpu.sync_copy(idx_hbm.at[wid], idx_v)
        pltpu.sync_copy(bias_hbm, bias_v)

        def fire(t):
            ireg = idx_v[t // 64, (t % 64) // 8, pl.ds((t % 8) * 16, 16)]
            pltpu.async_copy(z_hbm.at[ireg],
                             zbuf.at[t % 2], sem.at[t % 2])

        fire(0)

        def step(t, _):
            c, s = t // S, t % S
            pb = t % 2

            pltpu.make_async_copy(z_hbm.at[pl.ds(0, VC)],
                                  zbuf.at[pb], sem.at[pb]).wait()

            @pl.when(t + 1 < T)
            def _():
                fire(t + 1)

            NJ = ROW // 16

            @pl.when(s == 0)
            def _():
                def cp(j, _):
                    o = j * 16
                    for i in range(VC):
                        acc[i * 16 + j // 8, pl.ds((j % 8) * 16, 16)] = (
                            zbuf[pb, i, pl.ds(o, 16)])
                    return 0
                lax.fori_loop(0, NJ, cp, 0)

            @pl.when(s > 0)
            def _():
                def ad(j, _):
                    o = j * 16
                    for i in range(VC):
                        plsc.addupdate(
                            acc.at[i * 16 + j // 8, pl.ds((j % 8) * 16, 16)],
                            zbuf[pb, i, pl.ds(o, 16)])
                    return 0
                lax.fori_loop(0, NJ, ad, 0)

            @pl.when(s == S - 1)
            def _():
                def fin(j, _):
                    o = j * 16
                    bv = bias_v[0, pl.ds(o, 16)]
                    for i in range(VC):
                        r = i * 16 + j // 8
                        ds = pl.ds((j % 8) * 16, 16)
                        acc[r, ds] = jnp.maximum(acc[r, ds] + bv, 0.0)
                    return 0
                lax.fori_loop(0, NJ, fin, 0)
                v0 = wid * VPW + jnp.minimum(c * VC, VPW - VC)
                pltpu.sync_copy(acc, out_hbm.at[pl.ds(v0 * B, VC * B)])
            return 0

        lax.fori_loop(0, T, step, 0)

    return sc_gather


def kernel(x, trans_row, trans_col, trans_val, spiral_idx, W, b):
    B, N_IN, C = x.shape
    N_OUT, S = spiral_idx.shape
    C_OUT = W.shape[1]
    ROW = B * C_OUT
    NW = 32
    VC = 16
    VPW = N_OUT // NW
    NCH = -(-VPW // VC)
    T = NCH * S
    T_PAD = -(-T // 64) * 64
    NTI = T_PAD // 64

    NNZ = trans_row.shape[0]
    NNZP = -(-NNZ // 2048) * 2048
    pad = NNZP - NNZ
    colp = jnp.pad(trans_col.astype(jnp.int32), (0, pad))
    rowp = jnp.pad(trans_row.astype(jnp.int32), (0, pad))
    valp = jnp.pad(trans_val.astype(jnp.float32), (0, pad))
    x2d = x.reshape(B * N_IN, C)
    vxp = jnp.broadcast_to(valp[:, None], (NNZP, 16))
    pool = _make_sc_pool(B, N_IN, C, NNZP)
    pooled = pool(x2d, colp, rowp, valp, vxp).reshape(B, N_OUT, C)

    Ws = W.reshape(S, C, C_OUT).astype(jnp.bfloat16)
    Z = _matmul_z(pooled.astype(jnp.bfloat16), Ws)
    z2d = Z.reshape(S * N_OUT, ROW)

    starts = jnp.minimum(jnp.arange(NCH, dtype=jnp.int32) * VC, VPW - VC)
    vglob = (jnp.arange(NW, dtype=jnp.int32)[:, None, None] * VPW
             + starts[None, :, None]
             + jnp.arange(VC, dtype=jnp.int32)[None, None, :])
    idx_g = spiral_idx.astype(jnp.int32)[vglob.reshape(-1)]
    idx_g = (idx_g + jnp.arange(S, dtype=jnp.int32)[None, :] * N_OUT)
    idx_wp = (idx_g.reshape(NW, NCH, VC, S)
              .transpose(0, 1, 3, 2)
              .reshape(NW, T, VC))
    idx_wp = jnp.pad(idx_wp, ((0, 0), (0, T_PAD - T), (0, 0)))
    idx_wp = idx_wp.reshape(NW, NTI, 8, 128)
    bias_row = jnp.tile(b.astype(jnp.float32), B)[None, :]

    sc = _make_sc_gather(S, N_OUT, B, C_OUT, NW, VPW, NCH, NTI)
    out2 = sc(z2d, idx_wp, bias_row)
    return out2.reshape(N_OUT, B, C_OUT).transpose(1, 0, 2)
